# baseline (speedup 1.0000x reference)
import jax
import jax.numpy as jnp
from jax.experimental import pallas as pl
from jax.experimental.pallas import tpu as pltpu

_N = 10000
_D_IN = 128
_D_H = 32
_D_OUT = 7
_D_CAT = _D_IN + _D_H
_BLK = 3840


def _cell_body(x_ref, hT_ref, w_ref, outT_ref, HT_ref):
    x = x_ref[...]
    hT = hT_ref[...]

    wx_all = w_ref[0:_D_IN, 0:96]
    wzr = w_ref[_D_IN:_D_CAT, 0:64]
    whh = w_ref[_D_IN:_D_CAT, 64:96]
    wl = w_ref[_D_IN:_D_CAT, 96:96 + _D_OUT]
    bz = w_ref[_D_CAT:_D_CAT + _D_H, 0:1]
    br = w_ref[_D_CAT:_D_CAT + _D_H, 1:2]
    bh = w_ref[_D_CAT:_D_CAT + _D_H, 2:3]
    bl = w_ref[_D_CAT:_D_CAT + _D_OUT, 3:4]

    gx = jax.lax.dot_general(
        wx_all, x, (((0,), (1,)), ((), ())),
        preferred_element_type=jnp.float32)
    gzr = jax.lax.dot_general(
        wzr, hT, (((0,), (0,)), ((), ())),
        preferred_element_type=jnp.float32)

    z = jax.nn.sigmoid(gx[0:32] + gzr[0:32] + bz)
    r = jax.nn.sigmoid(gx[32:64] + gzr[32:64] + br)
    hr = hT * r
    ghh = jax.lax.dot_general(
        whh, hr, (((0,), (0,)), ((), ())),
        preferred_element_type=jnp.float32)
    ht = jnp.tanh(gx[64:96] + ghh + bh)
    HT = z * hT + (1.0 - z) * ht
    HT_ref[...] = HT
    outT_ref[...] = jax.lax.dot_general(
        wl, jnp.maximum(HT, 0.0), (((0,), (0,)), ((), ())),
        preferred_element_type=jnp.float32) + bl


def kernel(x, edge_index, edge_weight, h, W_z, b_z, W_r, b_r, W_h, b_h,
           W_lin, b_lin):
    del edge_index, edge_weight

    wz = W_z[0, 0] + W_z[1, 0]
    wr = W_r[0, 0] + W_r[1, 0]
    wh = W_h[0, 0] + W_h[1, 0]
    wl = jnp.pad(W_lin, ((_D_IN, 0), (0, 0)))
    wtop = jnp.concatenate([wz, wr, wh, wl], axis=1)
    wtop = jnp.pad(wtop, ((0, 0), (0, 128 - 103)))
    bl = jnp.pad(b_lin, (0, _D_H - _D_OUT))
    brow = jnp.stack([b_z, b_r, b_h, bl], axis=1)
    brow = jnp.pad(brow, ((0, 0), (0, 124)))
    wpack = jnp.concatenate([wtop, brow], axis=0)

    hT = h.T

    grid = (pl.cdiv(_N, _BLK),)
    col_spec = lambda d: pl.BlockSpec((d, _BLK), lambda i: (0, i))

    outT, HT = pl.pallas_call(
        _cell_body,
        grid=grid,
        in_specs=[
            pl.BlockSpec((_BLK, _D_IN), lambda i: (i, 0)),
            col_spec(_D_H),
            pl.BlockSpec((192, 128), lambda i: (0, 0)),
        ],
        out_specs=[
            col_spec(_D_OUT),
            col_spec(_D_H),
        ],
        out_shape=[
            jax.ShapeDtypeStruct((_D_OUT, _N), jnp.float32),
            jax.ShapeDtypeStruct((_D_H, _N), jnp.float32),
        ],
        compiler_params=pltpu.CompilerParams(
            dimension_semantics=("parallel",),
        ),
    )(x, hT, wpack)
    return outT.T, HT.T


# speedup vs baseline: 1.0622x; 1.0622x over previous
import jax
import jax.numpy as jnp
from jax.experimental import pallas as pl
from jax.experimental.pallas import tpu as pltpu

_N = 10000
_D_IN = 128
_D_H = 32
_D_OUT = 7
_D_CAT = _D_IN + _D_H
_BLK = 5120


def _cell_body(x_ref, hT_ref, w_ref, outT_ref, HT_ref):
    x = x_ref[...]
    hT = hT_ref[...]

    wx_all = w_ref[0:_D_IN, 0:96]
    wzr = w_ref[_D_IN:_D_CAT, 0:64]
    whh = w_ref[_D_IN:_D_CAT, 64:96]
    wl = w_ref[_D_IN:_D_CAT, 96:96 + _D_OUT]
    bz = w_ref[_D_CAT:_D_CAT + _D_H, 0:1]
    br = w_ref[_D_CAT:_D_CAT + _D_H, 1:2]
    bh = w_ref[_D_CAT:_D_CAT + _D_H, 2:3]
    bl = w_ref[_D_CAT:_D_CAT + _D_OUT, 3:4]

    gx = jax.lax.dot_general(
        wx_all, x, (((0,), (1,)), ((), ())),
        preferred_element_type=jnp.float32)
    gzr = jax.lax.dot_general(
        wzr, hT, (((0,), (0,)), ((), ())),
        preferred_element_type=jnp.float32)

    z = jax.nn.sigmoid(gx[0:32] + gzr[0:32] + bz)
    r = jax.nn.sigmoid(gx[32:64] + gzr[32:64] + br)
    hr = hT * r
    ghh = jax.lax.dot_general(
        whh, hr, (((0,), (0,)), ((), ())),
        preferred_element_type=jnp.float32)
    ht = jnp.tanh(gx[64:96] + ghh + bh)
    HT = z * hT + (1.0 - z) * ht
    HT_ref[...] = HT
    outT_ref[...] = jax.lax.dot_general(
        wl, jnp.maximum(HT, 0.0), (((0,), (0,)), ((), ())),
        preferred_element_type=jnp.float32) + bl


def kernel(x, edge_index, edge_weight, h, W_z, b_z, W_r, b_r, W_h, b_h,
           W_lin, b_lin):
    del edge_index, edge_weight

    wz = W_z[0, 0] + W_z[1, 0]
    wr = W_r[0, 0] + W_r[1, 0]
    wh = W_h[0, 0] + W_h[1, 0]
    wl = jnp.pad(W_lin, ((_D_IN, 0), (0, 0)))
    wtop = jnp.concatenate([wz, wr, wh, wl], axis=1)
    bl = jnp.pad(b_lin, (0, _D_H - _D_OUT))
    brow = jnp.stack([b_z, b_r, b_h, bl], axis=1)
    wpack = (jnp.pad(wtop, ((0, 32), (0, 128 - 103)))
             + jnp.pad(brow, ((_D_CAT, 0), (0, 124))))

    hT = h.T

    grid = (pl.cdiv(_N, _BLK),)
    col_spec = lambda d: pl.BlockSpec((d, _BLK), lambda i: (0, i))

    outT, HT = pl.pallas_call(
        _cell_body,
        grid=grid,
        in_specs=[
            pl.BlockSpec((_BLK, _D_IN), lambda i: (i, 0)),
            col_spec(_D_H),
            pl.BlockSpec((192, 128), lambda i: (0, 0)),
        ],
        out_specs=[
            col_spec(_D_OUT),
            col_spec(_D_H),
        ],
        out_shape=[
            jax.ShapeDtypeStruct((_D_OUT, _N), jnp.float32),
            jax.ShapeDtypeStruct((_D_H, _N), jnp.float32),
        ],
        compiler_params=pltpu.CompilerParams(
            dimension_semantics=("parallel",),
        ),
    )(x, hT, wpack)
    return outT.T, HT.T
